# Initial kernel scaffold; baseline (speedup 1.0000x reference)
import jax
import jax.numpy as jnp
from jax.experimental import pallas as pl


def kernel(x, adjs, W1, b1, W2, b2, W3, b3):
    raise NotImplementedError("write your pallas kernel here")


# speedup vs baseline: 5.2813x; 5.2813x over previous
import functools

import jax
import jax.numpy as jnp
from jax import lax
from jax.experimental import pallas as pl
from jax.experimental.pallas import tpu as pltpu
from jax.experimental.pallas import tpu_sc as plsc

N = 10000
NP = 10240
E = 160000
D = 256
H = 128
DO = 64
R = 1024
SLAB = NP // 16

_mesh = plsc.VectorSubcoreMesh(core_axis_name="c", subcore_axis_name="s")


EPT = E // 16
DCH = 80
DIT = EPT // DCH


@functools.partial(
    pl.kernel,
    out_type=jax.ShapeDtypeStruct((2 * NP,), jnp.float32),
    mesh=_mesh,
    scratch_types=[
        pltpu.VMEM((DCH,), jnp.int32),
        pltpu.VMEM((DCH,), jnp.float32),
        pltpu.VMEM_SHARED((NP,), jnp.float32),
    ],
)
def _sc_degree(adjs_hbm, zeros1_hbm, deg_hbm, idx_v, ones_v, acc_sh):
    c = lax.axis_index("c")
    s = lax.axis_index("s")
    pltpu.sync_copy(zeros1_hbm.at[pl.ds(s * SLAB, SLAB)],
                    acc_sh.at[pl.ds(s * SLAB, SLAB)])
    for j in range(DCH // 16):
        ones_v[pl.ds(j * 16, 16)] = jnp.full((16,), 1.0, jnp.float32)
    plsc.subcore_barrier()

    def body(i, carry):
        off = c * E + s * EPT + i * DCH
        pltpu.sync_copy(adjs_hbm.at[pl.ds(off, DCH)], idx_v)
        pltpu.sync_copy(ones_v, acc_sh.at[idx_v], add=True)
        return carry

    lax.fori_loop(0, DIT, body, 0)
    plsc.subcore_barrier()
    pltpu.sync_copy(acc_sh.at[pl.ds(s * SLAB, SLAB)],
                    deg_hbm.at[pl.ds(c * NP + s * SLAB, SLAB)])


@functools.partial(
    pl.kernel,
    out_type=[jax.ShapeDtypeStruct((NP, H), jnp.float32)] * 2,
    mesh=_mesh,
    scratch_types=[
        pltpu.VMEM((DCH,), jnp.int32),
        pltpu.VMEM((DCH,), jnp.int32),
        pltpu.VMEM((DCH, H), jnp.float32),
        pltpu.VMEM_SHARED((NP, H), jnp.float32),
    ],
)
def _sc_agg_split(u0_hbm, u1_hbm, adjs_hbm, zeros_hbm, z0_hbm, z1_hbm,
                  src_v, dst_v, rows_v, acc_sh):
    c = lax.axis_index("c")
    s = lax.axis_index("s")
    pltpu.sync_copy(zeros_hbm.at[pl.ds(s * SLAB, SLAB)],
                    acc_sh.at[pl.ds(s * SLAB, SLAB)])
    plsc.subcore_barrier()

    def run(u_hbm, z_hbm):
        def body(i, carry):
            off = s * EPT + i * DCH
            pltpu.sync_copy(adjs_hbm.at[pl.ds(off, DCH)], src_v)
            pltpu.sync_copy(adjs_hbm.at[pl.ds(E + off, DCH)], dst_v)
            pltpu.sync_copy(u_hbm.at[src_v], rows_v)
            pltpu.sync_copy(rows_v, acc_sh.at[dst_v], add=True)
            return carry

        lax.fori_loop(0, DIT, body, 0)
        plsc.subcore_barrier()
        pltpu.sync_copy(acc_sh.at[pl.ds(s * SLAB, SLAB)],
                        z_hbm.at[pl.ds(s * SLAB, SLAB)])

    @pl.when(c == 0)
    def _():
        run(u0_hbm, z0_hbm)

    @pl.when(c == 1)
    def _():
        run(u1_hbm, z1_hbm)


EPC = E // 2
EPT3 = EPC // 16
CH3 = 40
IT3 = EPT3 // CH3


@functools.partial(
    pl.kernel,
    out_type=jax.ShapeDtypeStruct((2, NP, H), jnp.float32),
    mesh=_mesh,
    scratch_types=[
        pltpu.VMEM((CH3,), jnp.int32),
        pltpu.VMEM((CH3,), jnp.int32),
        pltpu.VMEM((CH3, H), jnp.float32),
        pltpu.VMEM_SHARED((NP, H), jnp.float32),
    ],
)
def _sc_agg_edge(u_hbm, adjs_hbm, zeros_hbm, z_hbm,
                 src_v, dst_v, rows_v, acc_sh):
    c = lax.axis_index("c")
    s = lax.axis_index("s")
    pltpu.sync_copy(zeros_hbm.at[pl.ds(s * SLAB, SLAB)],
                    acc_sh.at[pl.ds(s * SLAB, SLAB)])
    plsc.subcore_barrier()

    def body(i, carry):
        off = c * EPC + s * EPT3 + i * CH3
        pltpu.sync_copy(adjs_hbm.at[pl.ds(off, CH3)], src_v)
        pltpu.sync_copy(adjs_hbm.at[pl.ds(E + off, CH3)], dst_v)
        pltpu.sync_copy(u_hbm.at[src_v], rows_v)
        pltpu.sync_copy(rows_v, acc_sh.at[dst_v], add=True)
        return carry

    lax.fori_loop(0, IT3, body, 0)
    plsc.subcore_barrier()
    pltpu.sync_copy(acc_sh.at[pl.ds(s * SLAB, SLAB)],
                    z_hbm.at[c, pl.ds(s * SLAB, SLAB)])


def _first_body(x_ref, w_ref, dego_ref, u0_ref, u1_ref):
    so = lax.rsqrt(jnp.maximum(dego_ref[...], 1.0))
    u = jnp.dot(x_ref[...], w_ref[...], preferred_element_type=jnp.float32)
    u = u * so[:, None]
    u0_ref[...] = u[:, :H]
    u1_ref[...] = u[:, H:]


def _tc_first(x, W1, dego):
    return pl.pallas_call(
        _first_body,
        grid=(NP // R,),
        in_specs=[
            pl.BlockSpec((R, D), lambda i: (i, 0)),
            pl.BlockSpec((D, D), lambda i: (0, 0)),
            pl.BlockSpec((R,), lambda i: (i,)),
        ],
        out_specs=[
            pl.BlockSpec((R, H), lambda i: (i, 0)),
            pl.BlockSpec((R, H), lambda i: (i, 0)),
        ],
        out_shape=[jax.ShapeDtypeStruct((NP, H), jnp.float32)] * 2,
    )(x, W1, dego)


def _mid_body(z0_ref, z1_ref, w_ref, b_ref, degi_ref, dego_ref, u0_ref, u1_ref):
    si = lax.rsqrt(jnp.maximum(degi_ref[...], 1.0))[:, None]
    so = lax.rsqrt(jnp.maximum(dego_ref[...], 1.0))[:, None]
    b = b_ref[...]
    a0 = jnp.maximum(z0_ref[...] * si + b[:H][None, :], 0.0)
    a1 = jnp.maximum(z1_ref[...] * si + b[H:][None, :], 0.0)
    u = (jnp.dot(a0, w_ref[:H, :], preferred_element_type=jnp.float32)
         + jnp.dot(a1, w_ref[H:, :], preferred_element_type=jnp.float32))
    u = u * so
    u0_ref[...] = u[:, :H]
    u1_ref[...] = u[:, H:]


def _tc_mid(z0, z1, W, b, degi, dego):
    return pl.pallas_call(
        _mid_body,
        grid=(NP // R,),
        in_specs=[
            pl.BlockSpec((R, H), lambda i: (i, 0)),
            pl.BlockSpec((R, H), lambda i: (i, 0)),
            pl.BlockSpec((D, D), lambda i: (0, 0)),
            pl.BlockSpec((D,), lambda i: (0,)),
            pl.BlockSpec((R,), lambda i: (i,)),
            pl.BlockSpec((R,), lambda i: (i,)),
        ],
        out_specs=[
            pl.BlockSpec((R, H), lambda i: (i, 0)),
            pl.BlockSpec((R, H), lambda i: (i, 0)),
        ],
        out_shape=[jax.ShapeDtypeStruct((NP, H), jnp.float32)] * 2,
    )(z0, z1, W, b, degi, dego)


def _last_body(z0_ref, z1_ref, w_ref, b_ref, degi_ref, dego_ref, u_ref):
    si = lax.rsqrt(jnp.maximum(degi_ref[...], 1.0))[:, None]
    so = lax.rsqrt(jnp.maximum(dego_ref[...], 1.0))[:, None]
    b = b_ref[...]
    a0 = jnp.maximum(z0_ref[...] * si + b[:H][None, :], 0.0)
    a1 = jnp.maximum(z1_ref[...] * si + b[H:][None, :], 0.0)
    u = (jnp.dot(a0, w_ref[:H, :], preferred_element_type=jnp.float32)
         + jnp.dot(a1, w_ref[H:, :], preferred_element_type=jnp.float32))
    u = u * so
    u_ref[...] = jnp.concatenate([u, jnp.zeros_like(u)], axis=1)


def _tc_last(z0, z1, W3, b, degi, dego):
    return pl.pallas_call(
        _last_body,
        grid=(NP // R,),
        in_specs=[
            pl.BlockSpec((R, H), lambda i: (i, 0)),
            pl.BlockSpec((R, H), lambda i: (i, 0)),
            pl.BlockSpec((D, DO), lambda i: (0, 0)),
            pl.BlockSpec((D,), lambda i: (0,)),
            pl.BlockSpec((R,), lambda i: (i,)),
            pl.BlockSpec((R,), lambda i: (i,)),
        ],
        out_specs=pl.BlockSpec((R, H), lambda i: (i, 0)),
        out_shape=jax.ShapeDtypeStruct((NP, H), jnp.float32),
    )(z0, z1, W3, b, degi, dego)


def _fin_body(z3_ref, b3_ref, degi_ref, out_ref):
    si = lax.rsqrt(jnp.maximum(degi_ref[...], 1.0))[:, None]
    out_ref[...] = (z3_ref[0][:, :DO] + z3_ref[1][:, :DO]) * si + b3_ref[...][None, :]


def _tc_fin(z3, b3, degi):
    return pl.pallas_call(
        _fin_body,
        grid=(NP // R,),
        in_specs=[
            pl.BlockSpec((2, R, H), lambda i: (0, i, 0)),
            pl.BlockSpec((DO,), lambda i: (0,)),
            pl.BlockSpec((R,), lambda i: (i,)),
        ],
        out_specs=pl.BlockSpec((R, DO), lambda i: (i, 0)),
        out_shape=jax.ShapeDtypeStruct((NP, DO), jnp.float32),
    )(z3, b3, degi)


def kernel(x, adjs, W1, b1, W2, b2, W3, b3):
    xp = jnp.pad(x, ((0, NP - N), (0, 0)))
    zeros1 = jnp.zeros((NP,), jnp.float32)
    zerosH = jnp.zeros((NP, H), jnp.float32)

    adjs_flat = adjs.reshape(2 * E)
    deg = _sc_degree(adjs_flat, zeros1)
    dego = deg[:NP]
    degi = deg[NP:]

    u0, u1 = _tc_first(xp, W1, dego)
    z0, z1 = _sc_agg_split(u0, u1, adjs_flat, zerosH)
    u0, u1 = _tc_mid(z0, z1, W2, b1, degi, dego)
    z0, z1 = _sc_agg_split(u0, u1, adjs_flat, zerosH)
    u3 = _tc_last(z0, z1, W3, b2, degi, dego)
    z3 = _sc_agg_edge(u3, adjs_flat, zerosH)
    out = _tc_fin(z3, b3, degi)
    return out[:N]


# speedup vs baseline: 8.3709x; 1.5850x over previous
import functools

import jax
import jax.numpy as jnp
from jax import lax
from jax.experimental import pallas as pl
from jax.experimental.pallas import tpu as pltpu
from jax.experimental.pallas import tpu_sc as plsc

N = 10000
NP = 10240
E = 160000
D = 256
H = 128
DO = 64
R = 1024
SLAB = NP // 16
NBUF = 4

_mesh = plsc.VectorSubcoreMesh(core_axis_name="c", subcore_axis_name="s")


EPT = E // 16
DCH = 80
DIT = EPT // DCH


def _agg_pipeline(u_hbm, acc_sh, adjs_hbm, src_base, dst_base, src_vs,
                  dst_vs, rows_vs, isem, gsem, chunk, nchunks):
    groups = nchunks // NBUF

    def body(g, carry):
        idescs = []
        gdescs = []
        for j in range(NBUF):
            off = (g * NBUF + j) * chunk
            idescs.append(pltpu.async_copy(
                adjs_hbm.at[pl.ds(src_base + off, chunk)], src_vs[j], isem))
            idescs.append(pltpu.async_copy(
                adjs_hbm.at[pl.ds(dst_base + off, chunk)], dst_vs[j], isem))
        for d in idescs:
            d.wait()
        for j in range(NBUF):
            gdescs.append(pltpu.async_copy(
                u_hbm.at[src_vs[j]], rows_vs[j], gsem))
        for d in gdescs:
            d.wait()
        for j in range(NBUF):
            pltpu.sync_copy(rows_vs[j], acc_sh.at[dst_vs[j]], add=True)
        return carry

    lax.fori_loop(0, groups, body, 0)
    for t in range(nchunks - groups * NBUF):
        off = (groups * NBUF + t) * chunk
        pltpu.sync_copy(adjs_hbm.at[pl.ds(src_base + off, chunk)], src_vs[t])
        pltpu.sync_copy(adjs_hbm.at[pl.ds(dst_base + off, chunk)], dst_vs[t])
        pltpu.sync_copy(u_hbm.at[src_vs[t]], rows_vs[t])
        pltpu.sync_copy(rows_vs[t], acc_sh.at[dst_vs[t]], add=True)


@functools.partial(
    pl.kernel,
    out_type=jax.ShapeDtypeStruct((2 * NP,), jnp.float32),
    mesh=_mesh,
    scratch_types=[
        pltpu.VMEM((DCH,), jnp.int32),
        pltpu.VMEM((DCH,), jnp.float32),
        pltpu.VMEM_SHARED((NP,), jnp.float32),
    ],
)
def _sc_degree(adjs_hbm, zeros1_hbm, deg_hbm, idx_v, ones_v, acc_sh):
    c = lax.axis_index("c")
    s = lax.axis_index("s")
    pltpu.sync_copy(zeros1_hbm.at[pl.ds(s * SLAB, SLAB)],
                    acc_sh.at[pl.ds(s * SLAB, SLAB)])
    for j in range(DCH // 16):
        ones_v[pl.ds(j * 16, 16)] = jnp.full((16,), 1.0, jnp.float32)
    plsc.subcore_barrier()

    def body(i, carry):
        off = c * E + s * EPT + i * DCH
        pltpu.sync_copy(adjs_hbm.at[pl.ds(off, DCH)], idx_v)
        pltpu.sync_copy(ones_v, acc_sh.at[idx_v], add=True)
        return carry

    lax.fori_loop(0, DIT, body, 0)
    plsc.subcore_barrier()
    pltpu.sync_copy(acc_sh.at[pl.ds(s * SLAB, SLAB)],
                    deg_hbm.at[pl.ds(c * NP + s * SLAB, SLAB)])


@functools.partial(
    pl.kernel,
    out_type=[jax.ShapeDtypeStruct((NP, H), jnp.float32)] * 2,
    mesh=_mesh,
    scratch_types=(
        [pltpu.VMEM((DCH,), jnp.int32)] * (2 * NBUF)
        + [pltpu.VMEM((DCH, H), jnp.float32)] * NBUF
        + [pltpu.SemaphoreType.DMA] * 2
        + [pltpu.VMEM_SHARED((NP, H), jnp.float32)]
    ),
)
def _sc_agg_split(u0_hbm, u1_hbm, adjs_hbm, zeros_hbm, z0_hbm, z1_hbm,
                  *scratch):
    src_vs = scratch[0:NBUF]
    dst_vs = scratch[NBUF:2 * NBUF]
    rows_vs = scratch[2 * NBUF:3 * NBUF]
    isem, gsem = scratch[3 * NBUF], scratch[3 * NBUF + 1]
    acc_sh = scratch[3 * NBUF + 2]

    c = lax.axis_index("c")
    s = lax.axis_index("s")
    pltpu.sync_copy(zeros_hbm.at[pl.ds(s * SLAB, SLAB)],
                    acc_sh.at[pl.ds(s * SLAB, SLAB)])
    plsc.subcore_barrier()

    @pl.when(c == 0)
    def _():
        _agg_pipeline(u0_hbm, acc_sh, adjs_hbm, s * EPT, E + s * EPT, src_vs,
                      dst_vs, rows_vs, isem, gsem, DCH, DIT)

    @pl.when(c == 1)
    def _():
        _agg_pipeline(u1_hbm, acc_sh, adjs_hbm, s * EPT, E + s * EPT, src_vs,
                      dst_vs, rows_vs, isem, gsem, DCH, DIT)

    plsc.subcore_barrier()

    @pl.when(c == 0)
    def _():
        pltpu.sync_copy(acc_sh.at[pl.ds(s * SLAB, SLAB)],
                        z0_hbm.at[pl.ds(s * SLAB, SLAB)])

    @pl.when(c == 1)
    def _():
        pltpu.sync_copy(acc_sh.at[pl.ds(s * SLAB, SLAB)],
                        z1_hbm.at[pl.ds(s * SLAB, SLAB)])


def _first_body(x_ref, w_ref, dego_ref, u0_ref, u1_ref):
    so = lax.rsqrt(jnp.maximum(dego_ref[...], 1.0))
    u = jnp.dot(x_ref[...], w_ref[...], preferred_element_type=jnp.float32)
    u = u * so[:, None]
    u0_ref[...] = u[:, :H]
    u1_ref[...] = u[:, H:]


def _tc_first(x, W1, dego):
    return pl.pallas_call(
        _first_body,
        grid=(NP // R,),
        in_specs=[
            pl.BlockSpec((R, D), lambda i: (i, 0)),
            pl.BlockSpec((D, D), lambda i: (0, 0)),
            pl.BlockSpec((R,), lambda i: (i,)),
        ],
        out_specs=[
            pl.BlockSpec((R, H), lambda i: (i, 0)),
            pl.BlockSpec((R, H), lambda i: (i, 0)),
        ],
        out_shape=[jax.ShapeDtypeStruct((NP, H), jnp.float32)] * 2,
    )(x, W1, dego)


def _mid_body(z0_ref, z1_ref, w_ref, b_ref, degi_ref, dego_ref, u0_ref, u1_ref):
    si = lax.rsqrt(jnp.maximum(degi_ref[...], 1.0))[:, None]
    so = lax.rsqrt(jnp.maximum(dego_ref[...], 1.0))[:, None]
    b = b_ref[...]
    a0 = jnp.maximum(z0_ref[...] * si + b[:H][None, :], 0.0)
    a1 = jnp.maximum(z1_ref[...] * si + b[H:][None, :], 0.0)
    u = (jnp.dot(a0, w_ref[:H, :], preferred_element_type=jnp.float32)
         + jnp.dot(a1, w_ref[H:, :], preferred_element_type=jnp.float32))
    u = u * so
    u0_ref[...] = u[:, :H]
    u1_ref[...] = u[:, H:]


def _tc_mid(z0, z1, W, b, degi, dego):
    return pl.pallas_call(
        _mid_body,
        grid=(NP // R,),
        in_specs=[
            pl.BlockSpec((R, H), lambda i: (i, 0)),
            pl.BlockSpec((R, H), lambda i: (i, 0)),
            pl.BlockSpec((D, D), lambda i: (0, 0)),
            pl.BlockSpec((D,), lambda i: (0,)),
            pl.BlockSpec((R,), lambda i: (i,)),
            pl.BlockSpec((R,), lambda i: (i,)),
        ],
        out_specs=[
            pl.BlockSpec((R, H), lambda i: (i, 0)),
            pl.BlockSpec((R, H), lambda i: (i, 0)),
        ],
        out_shape=[jax.ShapeDtypeStruct((NP, H), jnp.float32)] * 2,
    )(z0, z1, W, b, degi, dego)


def _last_body(z0_ref, z1_ref, w_ref, b_ref, degi_ref, dego_ref, u_ref):
    si = lax.rsqrt(jnp.maximum(degi_ref[...], 1.0))[:, None]
    so = lax.rsqrt(jnp.maximum(dego_ref[...], 1.0))[:, None]
    b = b_ref[...]
    a0 = jnp.maximum(z0_ref[...] * si + b[:H][None, :], 0.0)
    a1 = jnp.maximum(z1_ref[...] * si + b[H:][None, :], 0.0)
    u = (jnp.dot(a0, w_ref[:H, :], preferred_element_type=jnp.float32)
         + jnp.dot(a1, w_ref[H:, :], preferred_element_type=jnp.float32))
    u = u * so
    u_ref[...] = jnp.concatenate([u, jnp.zeros_like(u)], axis=1)


def _tc_last(z0, z1, W3, b, degi, dego):
    return pl.pallas_call(
        _last_body,
        grid=(NP // R,),
        in_specs=[
            pl.BlockSpec((R, H), lambda i: (i, 0)),
            pl.BlockSpec((R, H), lambda i: (i, 0)),
            pl.BlockSpec((D, DO), lambda i: (0, 0)),
            pl.BlockSpec((D,), lambda i: (0,)),
            pl.BlockSpec((R,), lambda i: (i,)),
            pl.BlockSpec((R,), lambda i: (i,)),
        ],
        out_specs=pl.BlockSpec((R, H), lambda i: (i, 0)),
        out_shape=jax.ShapeDtypeStruct((NP, H), jnp.float32),
    )(z0, z1, W3, b, degi, dego)


def _fin_body(z3_ref, b3_ref, degi_ref, out_ref):
    si = lax.rsqrt(jnp.maximum(degi_ref[...], 1.0))[:, None]
    out_ref[...] = z3_ref[:, :DO] * si + b3_ref[...][None, :]


def _tc_fin(z3, b3, degi):
    return pl.pallas_call(
        _fin_body,
        grid=(NP // R,),
        in_specs=[
            pl.BlockSpec((R, H), lambda i: (i, 0)),
            pl.BlockSpec((DO,), lambda i: (0,)),
            pl.BlockSpec((R,), lambda i: (i,)),
        ],
        out_specs=pl.BlockSpec((R, DO), lambda i: (i, 0)),
        out_shape=jax.ShapeDtypeStruct((NP, DO), jnp.float32),
    )(z3, b3, degi)


def kernel(x, adjs, W1, b1, W2, b2, W3, b3):
    xp = jnp.pad(x, ((0, NP - N), (0, 0)))
    zeros1 = jnp.zeros((NP,), jnp.float32)
    zerosH = jnp.zeros((NP, H), jnp.float32)

    adjs_flat = adjs.reshape(2 * E)
    deg = _sc_degree(adjs_flat, zeros1)
    dego = deg[:NP]
    degi = deg[NP:]

    u0, u1 = _tc_first(xp, W1, dego)
    z0, z1 = _sc_agg_split(u0, u1, adjs_flat, zerosH)
    u0, u1 = _tc_mid(z0, z1, W2, b1, degi, dego)
    z0, z1 = _sc_agg_split(u0, u1, adjs_flat, zerosH)
    u3 = _tc_last(z0, z1, W3, b2, degi, dego)
    z3, _unused = _sc_agg_split(u3, u3, adjs_flat, zerosH)
    out = _tc_fin(z3, b3, degi)
    return out[:N]


# speedup vs baseline: 10.3620x; 1.2379x over previous
import functools

import jax
import jax.numpy as jnp
from jax import lax
from jax.experimental import pallas as pl
from jax.experimental.pallas import tpu as pltpu
from jax.experimental.pallas import tpu_sc as plsc

N = 10000
NP = 10240
E = 160000
D = 256
H = 128
DO = 64
R = 1024
SLAB = NP // 16
NBUF = 4

_mesh = plsc.VectorSubcoreMesh(core_axis_name="c", subcore_axis_name="s")


EPT = E // 16
DCH = 80
DIT = EPT // DCH


def _agg_pipeline(u_hbm, acc_sh, adjs_hbm, src_base, dst_base, src_vs,
                  dst_vs, rows_vs, isems, gsems, ssems, chunk, nchunks):
    groups = nchunks // NBUF

    def body(g, carry):
        idescs = []
        gdescs = []
        for j in range(NBUF):
            off = (g * NBUF + j) * chunk

            @pl.when(g > 0)
            def _():
                pltpu.make_async_copy(rows_vs[j], acc_sh.at[dst_vs[j]],
                                      ssems[j]).wait()

            idescs.append(pltpu.async_copy(
                adjs_hbm.at[pl.ds(src_base + off, chunk)], src_vs[j],
                isems[j]))
            idescs.append(pltpu.async_copy(
                adjs_hbm.at[pl.ds(dst_base + off, chunk)], dst_vs[j],
                isems[j]))
        for j in range(NBUF):
            idescs[2 * j].wait()
            gdescs.append(pltpu.async_copy(
                u_hbm.at[src_vs[j]], rows_vs[j], gsems[j]))
        for j in range(NBUF):
            idescs[2 * j + 1].wait()
            gdescs[j].wait()
            pltpu.async_copy(rows_vs[j], acc_sh.at[dst_vs[j]], ssems[j],
                             add=True)
        return carry

    lax.fori_loop(0, groups, body, 0)
    for j in range(NBUF):
        pltpu.make_async_copy(rows_vs[j], acc_sh.at[dst_vs[j]],
                              ssems[j]).wait()
    for t in range(nchunks - groups * NBUF):
        off = (groups * NBUF + t) * chunk
        pltpu.sync_copy(adjs_hbm.at[pl.ds(src_base + off, chunk)], src_vs[t])
        pltpu.sync_copy(adjs_hbm.at[pl.ds(dst_base + off, chunk)], dst_vs[t])
        pltpu.sync_copy(u_hbm.at[src_vs[t]], rows_vs[t])
        pltpu.sync_copy(rows_vs[t], acc_sh.at[dst_vs[t]], add=True)


@functools.partial(
    pl.kernel,
    out_type=jax.ShapeDtypeStruct((2 * NP,), jnp.float32),
    mesh=_mesh,
    scratch_types=(
        [pltpu.VMEM((DCH,), jnp.int32)] * NBUF
        + [pltpu.VMEM((DCH,), jnp.float32)]
        + [pltpu.SemaphoreType.DMA]
        + [pltpu.VMEM_SHARED((NP,), jnp.float32)]
    ),
)
def _sc_degree(adjs_hbm, zeros1_hbm, deg_hbm, *scratch):
    idx_vs = scratch[0:NBUF]
    ones_v = scratch[NBUF]
    isem = scratch[NBUF + 1]
    acc_sh = scratch[NBUF + 2]
    c = lax.axis_index("c")
    s = lax.axis_index("s")
    pltpu.sync_copy(zeros1_hbm.at[pl.ds(s * SLAB, SLAB)],
                    acc_sh.at[pl.ds(s * SLAB, SLAB)])
    for j in range(DCH // 16):
        ones_v[pl.ds(j * 16, 16)] = jnp.full((16,), 1.0, jnp.float32)
    plsc.subcore_barrier()

    def body(g, carry):
        descs = []
        for j in range(NBUF):
            off = c * E + s * EPT + (g * NBUF + j) * DCH
            descs.append(pltpu.async_copy(
                adjs_hbm.at[pl.ds(off, DCH)], idx_vs[j], isem))
        for j in range(NBUF):
            descs[j].wait()
            pltpu.sync_copy(ones_v, acc_sh.at[idx_vs[j]], add=True)
        return carry

    lax.fori_loop(0, DIT // NBUF, body, 0)
    for t in range(DIT - (DIT // NBUF) * NBUF):
        off = c * E + s * EPT + ((DIT // NBUF) * NBUF + t) * DCH
        pltpu.sync_copy(adjs_hbm.at[pl.ds(off, DCH)], idx_vs[t])
        pltpu.sync_copy(ones_v, acc_sh.at[idx_vs[t]], add=True)
    plsc.subcore_barrier()
    pltpu.sync_copy(acc_sh.at[pl.ds(s * SLAB, SLAB)],
                    deg_hbm.at[pl.ds(c * NP + s * SLAB, SLAB)])


@functools.partial(
    pl.kernel,
    out_type=[jax.ShapeDtypeStruct((NP, H), jnp.float32)] * 2,
    mesh=_mesh,
    scratch_types=(
        [pltpu.VMEM((DCH,), jnp.int32)] * (2 * NBUF)
        + [pltpu.VMEM((DCH, H), jnp.float32)] * NBUF
        + [pltpu.SemaphoreType.DMA] * (3 * NBUF)
        + [pltpu.VMEM_SHARED((NP, H), jnp.float32)]
    ),
)
def _sc_agg_split(u0_hbm, u1_hbm, adjs_hbm, zeros_hbm, z0_hbm, z1_hbm,
                  *scratch):
    src_vs = scratch[0:NBUF]
    dst_vs = scratch[NBUF:2 * NBUF]
    rows_vs = scratch[2 * NBUF:3 * NBUF]
    isems = scratch[3 * NBUF:4 * NBUF]
    gsems = scratch[4 * NBUF:5 * NBUF]
    ssems = scratch[5 * NBUF:6 * NBUF]
    acc_sh = scratch[6 * NBUF]

    c = lax.axis_index("c")
    s = lax.axis_index("s")
    pltpu.sync_copy(zeros_hbm.at[pl.ds(s * SLAB, SLAB)],
                    acc_sh.at[pl.ds(s * SLAB, SLAB)])
    plsc.subcore_barrier()

    @pl.when(c == 0)
    def _():
        _agg_pipeline(u0_hbm, acc_sh, adjs_hbm, s * EPT, E + s * EPT, src_vs,
                      dst_vs, rows_vs, isems, gsems, ssems, DCH, DIT)

    @pl.when(c == 1)
    def _():
        _agg_pipeline(u1_hbm, acc_sh, adjs_hbm, s * EPT, E + s * EPT, src_vs,
                      dst_vs, rows_vs, isems, gsems, ssems, DCH, DIT)

    plsc.subcore_barrier()

    @pl.when(c == 0)
    def _():
        pltpu.sync_copy(acc_sh.at[pl.ds(s * SLAB, SLAB)],
                        z0_hbm.at[pl.ds(s * SLAB, SLAB)])

    @pl.when(c == 1)
    def _():
        pltpu.sync_copy(acc_sh.at[pl.ds(s * SLAB, SLAB)],
                        z1_hbm.at[pl.ds(s * SLAB, SLAB)])


def _first_body(x_ref, w_ref, dego_ref, u0_ref, u1_ref):
    so = lax.rsqrt(jnp.maximum(dego_ref[...], 1.0))
    u = jnp.dot(x_ref[...], w_ref[...], preferred_element_type=jnp.float32)
    u = u * so[:, None]
    u0_ref[...] = u[:, :H]
    u1_ref[...] = u[:, H:]


def _tc_first(x, W1, dego):
    return pl.pallas_call(
        _first_body,
        grid=(NP // R,),
        in_specs=[
            pl.BlockSpec((R, D), lambda i: (i, 0)),
            pl.BlockSpec((D, D), lambda i: (0, 0)),
            pl.BlockSpec((R,), lambda i: (i,)),
        ],
        out_specs=[
            pl.BlockSpec((R, H), lambda i: (i, 0)),
            pl.BlockSpec((R, H), lambda i: (i, 0)),
        ],
        out_shape=[jax.ShapeDtypeStruct((NP, H), jnp.float32)] * 2,
    )(x, W1, dego)


def _mid_body(z0_ref, z1_ref, w_ref, b_ref, degi_ref, dego_ref, u0_ref, u1_ref):
    si = lax.rsqrt(jnp.maximum(degi_ref[...], 1.0))[:, None]
    so = lax.rsqrt(jnp.maximum(dego_ref[...], 1.0))[:, None]
    b = b_ref[...]
    a0 = jnp.maximum(z0_ref[...] * si + b[:H][None, :], 0.0)
    a1 = jnp.maximum(z1_ref[...] * si + b[H:][None, :], 0.0)
    u = (jnp.dot(a0, w_ref[:H, :], preferred_element_type=jnp.float32)
         + jnp.dot(a1, w_ref[H:, :], preferred_element_type=jnp.float32))
    u = u * so
    u0_ref[...] = u[:, :H]
    u1_ref[...] = u[:, H:]


def _tc_mid(z0, z1, W, b, degi, dego):
    return pl.pallas_call(
        _mid_body,
        grid=(NP // R,),
        in_specs=[
            pl.BlockSpec((R, H), lambda i: (i, 0)),
            pl.BlockSpec((R, H), lambda i: (i, 0)),
            pl.BlockSpec((D, D), lambda i: (0, 0)),
            pl.BlockSpec((D,), lambda i: (0,)),
            pl.BlockSpec((R,), lambda i: (i,)),
            pl.BlockSpec((R,), lambda i: (i,)),
        ],
        out_specs=[
            pl.BlockSpec((R, H), lambda i: (i, 0)),
            pl.BlockSpec((R, H), lambda i: (i, 0)),
        ],
        out_shape=[jax.ShapeDtypeStruct((NP, H), jnp.float32)] * 2,
    )(z0, z1, W, b, degi, dego)


def _last_body(z0_ref, z1_ref, w_ref, b_ref, degi_ref, dego_ref, u_ref):
    si = lax.rsqrt(jnp.maximum(degi_ref[...], 1.0))[:, None]
    so = lax.rsqrt(jnp.maximum(dego_ref[...], 1.0))[:, None]
    b = b_ref[...]
    a0 = jnp.maximum(z0_ref[...] * si + b[:H][None, :], 0.0)
    a1 = jnp.maximum(z1_ref[...] * si + b[H:][None, :], 0.0)
    u = (jnp.dot(a0, w_ref[:H, :], preferred_element_type=jnp.float32)
         + jnp.dot(a1, w_ref[H:, :], preferred_element_type=jnp.float32))
    u = u * so
    u_ref[...] = jnp.concatenate([u, jnp.zeros_like(u)], axis=1)


def _tc_last(z0, z1, W3, b, degi, dego):
    return pl.pallas_call(
        _last_body,
        grid=(NP // R,),
        in_specs=[
            pl.BlockSpec((R, H), lambda i: (i, 0)),
            pl.BlockSpec((R, H), lambda i: (i, 0)),
            pl.BlockSpec((D, DO), lambda i: (0, 0)),
            pl.BlockSpec((D,), lambda i: (0,)),
            pl.BlockSpec((R,), lambda i: (i,)),
            pl.BlockSpec((R,), lambda i: (i,)),
        ],
        out_specs=pl.BlockSpec((R, H), lambda i: (i, 0)),
        out_shape=jax.ShapeDtypeStruct((NP, H), jnp.float32),
    )(z0, z1, W3, b, degi, dego)


def _fin_body(z3_ref, b3_ref, degi_ref, out_ref):
    si = lax.rsqrt(jnp.maximum(degi_ref[...], 1.0))[:, None]
    out_ref[...] = z3_ref[:, :DO] * si + b3_ref[...][None, :]


def _tc_fin(z3, b3, degi):
    return pl.pallas_call(
        _fin_body,
        grid=(NP // R,),
        in_specs=[
            pl.BlockSpec((R, H), lambda i: (i, 0)),
            pl.BlockSpec((DO,), lambda i: (0,)),
            pl.BlockSpec((R,), lambda i: (i,)),
        ],
        out_specs=pl.BlockSpec((R, DO), lambda i: (i, 0)),
        out_shape=jax.ShapeDtypeStruct((NP, DO), jnp.float32),
    )(z3, b3, degi)


def kernel(x, adjs, W1, b1, W2, b2, W3, b3):
    xp = jnp.pad(x, ((0, NP - N), (0, 0)))
    zeros1 = jnp.zeros((NP,), jnp.float32)
    zerosH = jnp.zeros((NP, H), jnp.float32)

    adjs_flat = adjs.reshape(2 * E)
    deg = _sc_degree(adjs_flat, zeros1)
    dego = deg[:NP]
    degi = deg[NP:]

    u0, u1 = _tc_first(xp, W1, dego)
    z0, z1 = _sc_agg_split(u0, u1, adjs_flat, zerosH)
    u0, u1 = _tc_mid(z0, z1, W2, b1, degi, dego)
    z0, z1 = _sc_agg_split(u0, u1, adjs_flat, zerosH)
    u3 = _tc_last(z0, z1, W3, b2, degi, dego)
    z3, _unused = _sc_agg_split(u3, u3, adjs_flat, zerosH)
    out = _tc_fin(z3, b3, degi)
    return out[:N]


# speedup vs baseline: 11.3150x; 1.0920x over previous
import functools

import jax
import jax.numpy as jnp
from jax import lax
from jax.experimental import pallas as pl
from jax.experimental.pallas import tpu as pltpu
from jax.experimental.pallas import tpu_sc as plsc

N = 10000
NP = 10240
E = 160000
D = 256
H = 128
DO = 64
R = 1024
SLAB = NP // 16
NBUF = 4
ANB = 3

_mesh = plsc.VectorSubcoreMesh(core_axis_name="c", subcore_axis_name="s")


EPT = E // 16
DCH = 80
DIT = EPT // DCH


def _agg_pipeline(u_hbm, acc_sh, adjs_hbm, srcbuf, dst_base,
                  dst_vs, rows_vs, isems, gsems, ssems, chunk, nchunks):
    groups = nchunks // ANB

    def body(g, carry):
        idescs = []
        gdescs = []
        for j in range(ANB):
            off = (g * ANB + j) * chunk

            @pl.when(g > 0)
            def _():
                pltpu.make_async_copy(rows_vs[j], acc_sh.at[dst_vs[j]],
                                      ssems[j]).wait()

            idescs.append(pltpu.async_copy(
                adjs_hbm.at[pl.ds(dst_base + off, chunk)], dst_vs[j],
                isems[j]))
            gdescs.append(pltpu.async_copy(
                u_hbm.at[srcbuf.at[pl.ds(off, chunk)]], rows_vs[j],
                gsems[j]))
        for j in range(ANB):
            idescs[j].wait()
            gdescs[j].wait()
            pltpu.async_copy(rows_vs[j], acc_sh.at[dst_vs[j]], ssems[j],
                             add=True)
        return carry

    lax.fori_loop(0, groups, body, 0)
    for j in range(ANB):
        pltpu.make_async_copy(rows_vs[j], acc_sh.at[dst_vs[j]],
                              ssems[j]).wait()
    for t in range(nchunks - groups * ANB):
        off = (groups * ANB + t) * chunk
        pltpu.sync_copy(adjs_hbm.at[pl.ds(dst_base + off, chunk)], dst_vs[t])
        pltpu.sync_copy(u_hbm.at[srcbuf.at[pl.ds(off, chunk)]], rows_vs[t])
        pltpu.sync_copy(rows_vs[t], acc_sh.at[dst_vs[t]], add=True)


@functools.partial(
    pl.kernel,
    out_type=jax.ShapeDtypeStruct((2 * NP,), jnp.float32),
    mesh=_mesh,
    scratch_types=(
        [pltpu.VMEM((DCH,), jnp.int32)] * NBUF
        + [pltpu.VMEM((DCH,), jnp.float32)]
        + [pltpu.SemaphoreType.DMA]
        + [pltpu.VMEM_SHARED((NP,), jnp.float32)]
    ),
)
def _sc_degree(adjs_hbm, zeros1_hbm, deg_hbm, *scratch):
    idx_vs = scratch[0:NBUF]
    ones_v = scratch[NBUF]
    isem = scratch[NBUF + 1]
    acc_sh = scratch[NBUF + 2]
    c = lax.axis_index("c")
    s = lax.axis_index("s")
    pltpu.sync_copy(zeros1_hbm.at[pl.ds(s * SLAB, SLAB)],
                    acc_sh.at[pl.ds(s * SLAB, SLAB)])
    for j in range(DCH // 16):
        ones_v[pl.ds(j * 16, 16)] = jnp.full((16,), 1.0, jnp.float32)
    plsc.subcore_barrier()

    def body(g, carry):
        descs = []
        for j in range(NBUF):
            off = c * E + s * EPT + (g * NBUF + j) * DCH
            descs.append(pltpu.async_copy(
                adjs_hbm.at[pl.ds(off, DCH)], idx_vs[j], isem))
        for j in range(NBUF):
            descs[j].wait()
            pltpu.sync_copy(ones_v, acc_sh.at[idx_vs[j]], add=True)
        return carry

    lax.fori_loop(0, DIT // NBUF, body, 0)
    for t in range(DIT - (DIT // NBUF) * NBUF):
        off = c * E + s * EPT + ((DIT // NBUF) * NBUF + t) * DCH
        pltpu.sync_copy(adjs_hbm.at[pl.ds(off, DCH)], idx_vs[t])
        pltpu.sync_copy(ones_v, acc_sh.at[idx_vs[t]], add=True)
    plsc.subcore_barrier()
    pltpu.sync_copy(acc_sh.at[pl.ds(s * SLAB, SLAB)],
                    deg_hbm.at[pl.ds(c * NP + s * SLAB, SLAB)])


@functools.partial(
    pl.kernel,
    out_type=[jax.ShapeDtypeStruct((NP, H), jnp.float32)] * 2,
    mesh=_mesh,
    scratch_types=(
        [pltpu.VMEM((EPT,), jnp.int32)]
        + [pltpu.VMEM((DCH,), jnp.int32)] * ANB
        + [pltpu.VMEM((DCH, H), jnp.float32)] * ANB
        + [pltpu.SemaphoreType.DMA] * (3 * ANB)
        + [pltpu.VMEM_SHARED((NP, H), jnp.float32)]
    ),
)
def _sc_agg_split(u0_hbm, u1_hbm, adjs_hbm, zeros_hbm, z0_hbm, z1_hbm,
                  *scratch):
    srcbuf = scratch[0]
    dst_vs = scratch[1:1 + ANB]
    rows_vs = scratch[1 + ANB:1 + 2 * ANB]
    isems = scratch[1 + 2 * ANB:1 + 3 * ANB]
    gsems = scratch[1 + 3 * ANB:1 + 4 * ANB]
    ssems = scratch[1 + 4 * ANB:1 + 5 * ANB]
    acc_sh = scratch[1 + 5 * ANB]

    c = lax.axis_index("c")
    s = lax.axis_index("s")
    pltpu.sync_copy(zeros_hbm.at[pl.ds(s * SLAB, SLAB)],
                    acc_sh.at[pl.ds(s * SLAB, SLAB)])
    pltpu.sync_copy(adjs_hbm.at[pl.ds(s * EPT, EPT)], srcbuf)
    plsc.subcore_barrier()

    @pl.when(c == 0)
    def _():
        _agg_pipeline(u0_hbm, acc_sh, adjs_hbm, srcbuf, E + s * EPT,
                      dst_vs, rows_vs, isems, gsems, ssems, DCH, DIT)

    @pl.when(c == 1)
    def _():
        _agg_pipeline(u1_hbm, acc_sh, adjs_hbm, srcbuf, E + s * EPT,
                      dst_vs, rows_vs, isems, gsems, ssems, DCH, DIT)

    plsc.subcore_barrier()

    @pl.when(c == 0)
    def _():
        pltpu.sync_copy(acc_sh.at[pl.ds(s * SLAB, SLAB)],
                        z0_hbm.at[pl.ds(s * SLAB, SLAB)])

    @pl.when(c == 1)
    def _():
        pltpu.sync_copy(acc_sh.at[pl.ds(s * SLAB, SLAB)],
                        z1_hbm.at[pl.ds(s * SLAB, SLAB)])


def _first_body(x_ref, w_ref, dego_ref, u0_ref, u1_ref):
    so = lax.rsqrt(jnp.maximum(dego_ref[...], 1.0))
    u = jnp.dot(x_ref[...], w_ref[...], preferred_element_type=jnp.float32)
    u = u * so[:, None]
    u0_ref[...] = u[:, :H]
    u1_ref[...] = u[:, H:]


def _tc_first(x, W1, dego):
    return pl.pallas_call(
        _first_body,
        grid=(NP // R,),
        in_specs=[
            pl.BlockSpec((R, D), lambda i: (i, 0)),
            pl.BlockSpec((D, D), lambda i: (0, 0)),
            pl.BlockSpec((R,), lambda i: (i,)),
        ],
        out_specs=[
            pl.BlockSpec((R, H), lambda i: (i, 0)),
            pl.BlockSpec((R, H), lambda i: (i, 0)),
        ],
        out_shape=[jax.ShapeDtypeStruct((NP, H), jnp.float32)] * 2,
    )(x, W1, dego)


def _mid_body(z0_ref, z1_ref, w_ref, b_ref, degi_ref, dego_ref, u0_ref, u1_ref):
    si = lax.rsqrt(jnp.maximum(degi_ref[...], 1.0))[:, None]
    so = lax.rsqrt(jnp.maximum(dego_ref[...], 1.0))[:, None]
    b = b_ref[...]
    a0 = jnp.maximum(z0_ref[...] * si + b[:H][None, :], 0.0)
    a1 = jnp.maximum(z1_ref[...] * si + b[H:][None, :], 0.0)
    u = (jnp.dot(a0, w_ref[:H, :], preferred_element_type=jnp.float32)
         + jnp.dot(a1, w_ref[H:, :], preferred_element_type=jnp.float32))
    u = u * so
    u0_ref[...] = u[:, :H]
    u1_ref[...] = u[:, H:]


def _tc_mid(z0, z1, W, b, degi, dego):
    return pl.pallas_call(
        _mid_body,
        grid=(NP // R,),
        in_specs=[
            pl.BlockSpec((R, H), lambda i: (i, 0)),
            pl.BlockSpec((R, H), lambda i: (i, 0)),
            pl.BlockSpec((D, D), lambda i: (0, 0)),
            pl.BlockSpec((D,), lambda i: (0,)),
            pl.BlockSpec((R,), lambda i: (i,)),
            pl.BlockSpec((R,), lambda i: (i,)),
        ],
        out_specs=[
            pl.BlockSpec((R, H), lambda i: (i, 0)),
            pl.BlockSpec((R, H), lambda i: (i, 0)),
        ],
        out_shape=[jax.ShapeDtypeStruct((NP, H), jnp.float32)] * 2,
    )(z0, z1, W, b, degi, dego)


def _last_body(z0_ref, z1_ref, w_ref, b_ref, degi_ref, dego_ref, u_ref):
    si = lax.rsqrt(jnp.maximum(degi_ref[...], 1.0))[:, None]
    so = lax.rsqrt(jnp.maximum(dego_ref[...], 1.0))[:, None]
    b = b_ref[...]
    a0 = jnp.maximum(z0_ref[...] * si + b[:H][None, :], 0.0)
    a1 = jnp.maximum(z1_ref[...] * si + b[H:][None, :], 0.0)
    u = (jnp.dot(a0, w_ref[:H, :], preferred_element_type=jnp.float32)
         + jnp.dot(a1, w_ref[H:, :], preferred_element_type=jnp.float32))
    u = u * so
    u_ref[...] = jnp.concatenate([u, jnp.zeros_like(u)], axis=1)


def _tc_last(z0, z1, W3, b, degi, dego):
    return pl.pallas_call(
        _last_body,
        grid=(NP // R,),
        in_specs=[
            pl.BlockSpec((R, H), lambda i: (i, 0)),
            pl.BlockSpec((R, H), lambda i: (i, 0)),
            pl.BlockSpec((D, DO), lambda i: (0, 0)),
            pl.BlockSpec((D,), lambda i: (0,)),
            pl.BlockSpec((R,), lambda i: (i,)),
            pl.BlockSpec((R,), lambda i: (i,)),
        ],
        out_specs=pl.BlockSpec((R, H), lambda i: (i, 0)),
        out_shape=jax.ShapeDtypeStruct((NP, H), jnp.float32),
    )(z0, z1, W3, b, degi, dego)


def _fin_body(z3_ref, b3_ref, degi_ref, out_ref):
    si = lax.rsqrt(jnp.maximum(degi_ref[...], 1.0))[:, None]
    out_ref[...] = z3_ref[:, :DO] * si + b3_ref[...][None, :]


def _tc_fin(z3, b3, degi):
    return pl.pallas_call(
        _fin_body,
        grid=(NP // R,),
        in_specs=[
            pl.BlockSpec((R, H), lambda i: (i, 0)),
            pl.BlockSpec((DO,), lambda i: (0,)),
            pl.BlockSpec((R,), lambda i: (i,)),
        ],
        out_specs=pl.BlockSpec((R, DO), lambda i: (i, 0)),
        out_shape=jax.ShapeDtypeStruct((NP, DO), jnp.float32),
    )(z3, b3, degi)


def kernel(x, adjs, W1, b1, W2, b2, W3, b3):
    xp = jnp.pad(x, ((0, NP - N), (0, 0)))
    zeros1 = jnp.zeros((NP,), jnp.float32)
    zerosH = jnp.zeros((NP, H), jnp.float32)

    adjs_flat = adjs.reshape(2 * E)
    deg = _sc_degree(adjs_flat, zeros1)
    dego = deg[:NP]
    degi = deg[NP:]

    u0, u1 = _tc_first(xp, W1, dego)
    z0, z1 = _sc_agg_split(u0, u1, adjs_flat, zerosH)
    u0, u1 = _tc_mid(z0, z1, W2, b1, degi, dego)
    z0, z1 = _sc_agg_split(u0, u1, adjs_flat, zerosH)
    u3 = _tc_last(z0, z1, W3, b2, degi, dego)
    z3, _unused = _sc_agg_split(u3, u3, adjs_flat, zerosH)
    out = _tc_fin(z3, b3, degi)
    return out[:N]
